# baseline (speedup 1.0000x reference)
import jax
import jax.numpy as jnp
from jax import lax
from jax.experimental import pallas as pl
from jax.experimental.pallas import tpu as pltpu
from jax.experimental.pallas import tpu_sc as plsc

B, N, L, D = 1, 512, 1024, 64
DP = D + 1
V_MSA = 22
NC, NS = 2, 16
NW = NC * NS
N_PER_W = N // NW
NQ = 4
LT = 64
NLT = L // LT
N_CHUNKS = (N_PER_W // NQ) * NLT
NBUF = 3


def _body(msa_hbm, idx_hbm, emb_hbm, pe_hbm, peq_hbm, out_hbm,
          tbl, embv, peqv, idxv, stage, part, msav, obuf, pe_loc, pe_sh,
          sem_g, sem_p, sem_l, sem_o0, sem_o1, sem_o2):
    sid = lax.axis_index("s")
    wid = sid * NC + lax.axis_index("c")
    n0 = wid * N_PER_W
    ls = sid * LT

    pltpu.sync_copy(idx_hbm.at[0], idxv)
    gat = pltpu.async_copy(pe_hbm.at[idxv.at[pl.ds(ls, LT)]], stage, sem_g)

    pltpu.sync_copy(msa_hbm.at[0, pl.ds(n0, N_PER_W)], msav)
    pltpu.sync_copy(emb_hbm, embv)
    pltpu.sync_copy(peq_hbm, peqv)

    peq_regs = [[peqv[s, pl.ds(16 * j, 16)] for j in range(4)] for s in range(2)]
    for s in range(2):
        for i in range(V_MSA):
            for j in range(4):
                tbl[pl.ds((s * V_MSA + i) * DP + 16 * j, 16)] = (
                    embv[i, pl.ds(16 * j, 16)] + peq_regs[s][j])

    col16 = lax.iota(jnp.int32, 16)
    cvecs = [(col16 + 16 * j) * LT for j in range(4)]
    gat.wait()

    def transpose_row(r, _):
        for j in range(4):
            plsc.store_scatter(part, [cvecs[j] + r], stage[r, pl.ds(16 * j, 16)])
        return 0

    lax.fori_loop(0, LT, transpose_row, 0)

    pubs = [pltpu.async_copy(part.at[pl.ds(d * LT, LT)],
                             pe_sh.at[sid, d], sem_p)
            for d in range(D)]
    for p in pubs:
        p.wait()
    plsc.subcore_barrier()

    sem_o = [sem_o0, sem_o1, sem_o2]

    def load_petile(lt, pb):
        return pltpu.async_copy(pe_sh.at[lt], pe_loc.at[pb], sem_l)

    def issue_out(cc, b):
        ng = n0 + (cc % (N_PER_W // NQ)) * NQ
        l0 = (cc // (N_PER_W // NQ)) * LT
        pltpu.async_copy(obuf.at[b],
                         out_hbm.at[pl.ds(ng, NQ), :, pl.ds(l0, LT)],
                         sem_o[b])

    def drain_out(b):
        pltpu.make_async_copy(obuf.at[b],
                              out_hbm.at[pl.ds(0, NQ), :, pl.ds(0, LT)],
                              sem_o[b]).wait()

    def compute(cc, b, pb):
        nq = (cc % (N_PER_W // NQ)) * NQ
        l0 = (cc // (N_PER_W // NQ)) * LT
        offs = [jnp.where(n0 + nq + nn == 0, 0, V_MSA).astype(jnp.int32)
                for nn in range(NQ)]

        def lblock(lb, _):
            lbase = l0 + lb * 16
            tvecs = [(msav[nq + nn, pl.ds(lbase, 16)] + offs[nn]) * DP
                     for nn in range(NQ)]
            for d0 in range(0, D, 4):
                ps = [pe_loc[pb, d0 + i, pl.ds(lb * 16, 16)] for i in range(4)]
                gs = [[plsc.load_gather(tbl, [tvecs[nn] + (d0 + i)])
                       for nn in range(NQ)] for i in range(4)]
                for i in range(4):
                    for nn in range(NQ):
                        obuf[b, nn, d0 + i, pl.ds(lb * 16, 16)] = (
                            gs[i][nn] + ps[i])
            return 0

        lax.fori_loop(0, LT // 16, lblock, 0)

    load_petile(jnp.int32(0), 0).wait()
    lp_cur = 0

    compute(jnp.int32(0), 0, lp_cur)
    issue_out(jnp.int32(0), 0)
    compute(jnp.int32(1), 1, lp_cur)
    issue_out(jnp.int32(1), 1)
    compute(jnp.int32(2), 2, lp_cur)
    issue_out(jnp.int32(2), 2)

    def outer(co, lp):
        for j in range(NBUF):
            cc = co * NBUF + j
            b = j
            lt = cc // NQ
            start_pf = jnp.logical_and(cc % NQ == NQ - 1, lt + 1 < NLT)
            @pl.when(start_pf)
            def _():
                load_petile(lt + 1, 1 - lp)

            drain_out(b)
            pb = jnp.where(cc % NQ == 0, 1 - lp, lp)
            @pl.when(cc % NQ == 0)
            def _():
                pltpu.make_async_copy(pe_sh.at[0], pe_loc.at[0],
                                      sem_l).wait()
            lp = jnp.where(cc % NQ == 0, 1 - lp, lp)
            compute(cc, b, pb)
            issue_out(cc, b)
        return lp

    lp_fin = lax.fori_loop(1, N_CHUNKS // NBUF, outer, jnp.int32(lp_cur))

    drain_out(0)
    compute(jnp.int32(N_CHUNKS - 1), 0, lp_fin)
    issue_out(jnp.int32(N_CHUNKS - 1), 0)
    drain_out(1)
    drain_out(2)
    drain_out(0)


@jax.jit
def kernel(msa, idx, emb_W, pe_buf, pe_q):
    mesh = plsc.VectorSubcoreMesh(core_axis_name="c", subcore_axis_name="s",
                                  num_cores=NC, num_subcores=NS)
    fn = pl.kernel(
        _body,
        out_type=jax.ShapeDtypeStruct((N, D, L), jnp.float32),
        mesh=mesh,
        scratch_types=[
            pltpu.VMEM((2 * V_MSA * DP,), jnp.float32),
            pltpu.VMEM((V_MSA, D), jnp.float32),
            pltpu.VMEM((2, D), jnp.float32),
            pltpu.VMEM((L,), jnp.int32),
            pltpu.VMEM((LT, D), jnp.float32),
            pltpu.VMEM((D * LT,), jnp.float32),
            pltpu.VMEM((N_PER_W, L), jnp.int32),
            pltpu.VMEM((NBUF, NQ, D, LT), jnp.float32),
            pltpu.VMEM((2, D, LT), jnp.float32),
            pltpu.VMEM_SHARED((NLT, D, LT), jnp.float32),
            pltpu.SemaphoreType.DMA,
            pltpu.SemaphoreType.DMA,
            pltpu.SemaphoreType.DMA,
            pltpu.SemaphoreType.DMA,
            pltpu.SemaphoreType.DMA,
            pltpu.SemaphoreType.DMA,
        ],
        compiler_params=pltpu.CompilerParams(needs_layout_passes=False,
                                             use_tc_tiling_on_sc=False),
    )
    out_ndl = fn(msa, idx, emb_W, pe_buf, pe_q)
    return jnp.swapaxes(out_ndl, 1, 2)[None]


# speedup vs baseline: 1.0532x; 1.0532x over previous
import jax
import jax.numpy as jnp
from jax import lax
from jax.experimental import pallas as pl
from jax.experimental.pallas import tpu as pltpu
from jax.experimental.pallas import tpu_sc as plsc

B, N, L, D = 1, 512, 1024, 64
DP = D + 1
V_MSA = 22
NC, NS = 2, 16
NW = NC * NS
N_PER_W = N // NW
NQ = 2
LT = 128
LS = L // NS
NLT = L // LT
CPT = N_PER_W // NQ
N_CHUNKS = CPT * NLT
NBUF = 3


def _body(msa_hbm, idx_hbm, emb_hbm, pe_hbm, peq_hbm, out_hbm,
          tbl, embv, peqv, idxv, stage, part, msav, obuf, pe_loc, pe_sh,
          sem_g, sem_p, sem_l, sem_o0, sem_o1, sem_o2):
    sid = lax.axis_index("s")
    wid = sid * NC + lax.axis_index("c")
    n0 = wid * N_PER_W
    ls = sid * LS

    pltpu.sync_copy(idx_hbm.at[0], idxv)
    gat = pltpu.async_copy(pe_hbm.at[idxv.at[pl.ds(ls, LS)]], stage, sem_g)

    pltpu.sync_copy(msa_hbm.at[0, pl.ds(n0, N_PER_W)], msav)
    pltpu.sync_copy(emb_hbm, embv)
    pltpu.sync_copy(peq_hbm, peqv)

    peq_regs = [[peqv[s, pl.ds(16 * j, 16)] for j in range(4)] for s in range(2)]
    for s in range(2):
        for i in range(V_MSA):
            for j in range(4):
                tbl[pl.ds((s * V_MSA + i) * DP + 16 * j, 16)] = (
                    embv[i, pl.ds(16 * j, 16)] + peq_regs[s][j])

    col16 = lax.iota(jnp.int32, 16)
    cvecs = [(col16 + 16 * j) * LS for j in range(4)]
    gat.wait()

    def transpose_row(r, _):
        for j in range(4):
            plsc.store_scatter(part, [cvecs[j] + r], stage[r, pl.ds(16 * j, 16)])
        return 0

    lax.fori_loop(0, LS, transpose_row, 0)

    pubs = [pltpu.async_copy(part.at[pl.ds(d * LS, LS)],
                             pe_sh.at[ls // LT, d, pl.ds(ls % LT, LS)], sem_p)
            for d in range(D)]
    for p in pubs:
        p.wait()
    plsc.subcore_barrier()

    sem_o = [sem_o0, sem_o1, sem_o2]

    def load_petile(lt, pb):
        return pltpu.async_copy(pe_sh.at[lt], pe_loc.at[pb], sem_l)

    def issue_out(cc, b):
        ng = n0 + (cc % CPT) * NQ
        l0 = (cc // CPT) * LT
        pltpu.async_copy(obuf.at[b],
                         out_hbm.at[pl.ds(ng, NQ), :, pl.ds(l0, LT)],
                         sem_o[b])

    def drain_out(b):
        pltpu.make_async_copy(obuf.at[b],
                              out_hbm.at[pl.ds(0, NQ), :, pl.ds(0, LT)],
                              sem_o[b]).wait()

    def compute(cc, b, pb):
        nq = (cc % CPT) * NQ
        l0 = (cc // CPT) * LT
        offs = [jnp.where(n0 + nq + nn == 0, 0, V_MSA).astype(jnp.int32)
                for nn in range(NQ)]

        def lblock(lb, _):
            lbase = l0 + lb * 16
            tvecs = [(msav[nq + nn, pl.ds(lbase, 16)] + offs[nn]) * DP
                     for nn in range(NQ)]
            for d0 in range(0, D, 4):
                ps = [pe_loc[pb, d0 + i, pl.ds(lb * 16, 16)] for i in range(4)]
                gs = [[plsc.load_gather(tbl, [tvecs[nn] + (d0 + i)])
                       for nn in range(NQ)] for i in range(4)]
                for i in range(4):
                    for nn in range(NQ):
                        obuf[b, nn, d0 + i, pl.ds(lb * 16, 16)] = (
                            gs[i][nn] + ps[i])
            return 0

        lax.fori_loop(0, LT // 16, lblock, 0)

    load_petile(jnp.int32(0), 0).wait()
    lp_cur = 0

    compute(jnp.int32(0), 0, lp_cur)
    issue_out(jnp.int32(0), 0)
    compute(jnp.int32(1), 1, lp_cur)
    issue_out(jnp.int32(1), 1)
    compute(jnp.int32(2), 2, lp_cur)
    issue_out(jnp.int32(2), 2)

    def outer(co, lp):
        for j in range(NBUF):
            cc = co * NBUF + j
            b = j
            lt = cc // CPT
            start_pf = jnp.logical_and(cc % CPT == CPT - 1, lt + 1 < NLT)
            @pl.when(start_pf)
            def _():
                load_petile(lt + 1, 1 - lp)

            drain_out(b)
            pb = jnp.where(cc % CPT == 0, 1 - lp, lp)
            @pl.when(cc % CPT == 0)
            def _():
                pltpu.make_async_copy(pe_sh.at[0], pe_loc.at[0],
                                      sem_l).wait()
            lp = jnp.where(cc % CPT == 0, 1 - lp, lp)
            compute(cc, b, pb)
            issue_out(cc, b)
        return lp

    lp_fin = lax.fori_loop(1, N_CHUNKS // NBUF, outer, jnp.int32(lp_cur))

    drain_out(0)
    compute(jnp.int32(N_CHUNKS - 1), 0, lp_fin)
    issue_out(jnp.int32(N_CHUNKS - 1), 0)
    drain_out(1)
    drain_out(2)
    drain_out(0)


@jax.jit
def kernel(msa, idx, emb_W, pe_buf, pe_q):
    mesh = plsc.VectorSubcoreMesh(core_axis_name="c", subcore_axis_name="s",
                                  num_cores=NC, num_subcores=NS)
    fn = pl.kernel(
        _body,
        out_type=jax.ShapeDtypeStruct((N, D, L), jnp.float32),
        mesh=mesh,
        scratch_types=[
            pltpu.VMEM((2 * V_MSA * DP,), jnp.float32),
            pltpu.VMEM((V_MSA, D), jnp.float32),
            pltpu.VMEM((2, D), jnp.float32),
            pltpu.VMEM((L,), jnp.int32),
            pltpu.VMEM((LS, D), jnp.float32),
            pltpu.VMEM((D * LS,), jnp.float32),
            pltpu.VMEM((N_PER_W, L), jnp.int32),
            pltpu.VMEM((NBUF, NQ, D, LT), jnp.float32),
            pltpu.VMEM((2, D, LT), jnp.float32),
            pltpu.VMEM_SHARED((NLT, D, LT), jnp.float32),
            pltpu.SemaphoreType.DMA,
            pltpu.SemaphoreType.DMA,
            pltpu.SemaphoreType.DMA,
            pltpu.SemaphoreType.DMA,
            pltpu.SemaphoreType.DMA,
            pltpu.SemaphoreType.DMA,
        ],
        compiler_params=pltpu.CompilerParams(needs_layout_passes=False,
                                             use_tc_tiling_on_sc=False),
    )
    out_ndl = fn(msa, idx, emb_W, pe_buf, pe_q)
    return jnp.swapaxes(out_ndl, 1, 2)[None]
